# baseline (speedup 1.0000x reference)
import functools

import jax
import jax.numpy as jnp
from jax import lax
from jax.experimental import pallas as pl
from jax.experimental.pallas import tpu as pltpu
from jax.experimental.pallas import tpu_sc as plsc

EMB_DIM = 32
BATCH = 16384
NF = 26
B = BATCH * NF

NC = 2
NS = 16
NW = NC * NS

PER_TILE = B // NW
CHUNK = 256
CPT = PER_TILE // CHUNK
SUBS = CHUNK * EMB_DIM // 128


@jax.jit
def _gather_rows(idx, table4):
    mesh = plsc.VectorSubcoreMesh(core_axis_name="c", subcore_axis_name="s")

    @functools.partial(
        pl.kernel,
        out_type=jax.ShapeDtypeStruct((NF, BATCH * EMB_DIM // 128, 128),
                                      jnp.float32),
        mesh=mesh,
        scratch_types=[
            pltpu.VMEM((PER_TILE,), jnp.int32),
            pltpu.VMEM((CHUNK,), jnp.int32),
            pltpu.VMEM((CHUNK,), jnp.int32),
            pltpu.VMEM((CHUNK, 128), jnp.float32),
            pltpu.VMEM((CHUNK, 128), jnp.float32),
            pltpu.VMEM((SUBS, 128), jnp.float32),
            pltpu.VMEM((SUBS, 128), jnp.float32),
            pltpu.SemaphoreType.DMA,
            pltpu.SemaphoreType.DMA,
            pltpu.SemaphoreType.DMA,
            pltpu.SemaphoreType.DMA,
        ],
        compiler_params=pltpu.CompilerParams(
            use_tc_tiling_on_sc=False, needs_layout_passes=False
        ),
    )
    def k(idx_hbm, table_hbm, out_hbm, idx_v, q0, q1, r0, r1, o0, o1,
          g0, g1, w0, w1):
        wid = lax.axis_index("s") * NC + lax.axis_index("c")
        base = wid * PER_TILE
        pltpu.sync_copy(idx_hbm.at[pl.ds(base, PER_TILE)], idx_v)
        lanes = lax.iota(jnp.int32, 16)
        r_sc_base = lax.shift_right_logical(lanes, 2)
        c0_sc = lax.shift_left(lax.bitwise_and(lanes, 3), 5)

        def q_fill(c, qv):
            off = c * CHUNK

            def body(i, carry):
                v = idx_v[pl.ds(off + i * 16, 16)]
                qv[pl.ds(i * 16, 16)] = lax.shift_right_logical(v, 2)
                return carry

            lax.fori_loop(0, CHUNK // 16, body, 0)

        def gather(qv, rv, sem):
            return pltpu.async_copy(table_hbm.at[qv], rv, sem)

        def extract(c, rv, ov):
            off = c * CHUNK

            def jb_body(jb, carry):
                v = idx_v[pl.ds(off + jb * 16, 16)]
                col0 = lax.shift_left(lax.bitwise_and(v, 3), 5)
                row_g = jb * 16 + lanes
                r_sc = jb * 4 + r_sc_base
                for d in range(EMB_DIM):
                    vals = plsc.load_gather(rv, [row_g, col0 + d])
                    plsc.store_scatter(ov, [r_sc, c0_sc + d], vals)
                return carry

            lax.fori_loop(0, CHUNK // 16, jb_body, 0)

        def writeback(c, ov, sem):
            p0 = base + c * CHUNK
            f = p0 // BATCH
            sub0 = (p0 % BATCH) // 4
            return pltpu.async_copy(ov, out_hbm.at[f, pl.ds(sub0, SUBS)], sem)

        def pair_body(c2, carry):
            c = c2 * 2
            q_fill(c, q0)
            ga = gather(q0, r0, g0)
            q_fill(c + 1, q1)
            gb = gather(q1, r1, g1)
            ga.wait()
            extract(c, r0, o0)
            wa = writeback(c, o0, w0)
            gb.wait()
            extract(c + 1, r1, o1)
            wb = writeback(c + 1, o1, w1)
            wa.wait()
            wb.wait()
            return carry

        lax.fori_loop(0, CPT // 2, pair_body, 0)

    return k(idx, table4)


def kernel(cond_ids, emb_weight):
    idx = cond_ids.astype(jnp.int32).T.reshape(B)
    table4 = emb_weight.reshape(250000, 128)
    out = _gather_rows(idx, table4)
    return out.reshape(NF, BATCH, EMB_DIM).transpose(1, 0, 2)


# speedup vs baseline: 1.7065x; 1.7065x over previous
import functools

import jax
import jax.numpy as jnp
from jax import lax
from jax.experimental import pallas as pl
from jax.experimental.pallas import tpu as pltpu
from jax.experimental.pallas import tpu_sc as plsc

EMB_DIM = 32
BATCH = 16384
NF = 26

NC = 2
NS = 16
NW = NC * NS

CHUNK = 1024
NBLK = BATCH // CHUNK
N_UNITS = NF * NBLK
UPW = N_UNITS // NW


@jax.jit
def _gather_rows(idx, table):
    mesh = plsc.VectorSubcoreMesh(core_axis_name="c", subcore_axis_name="s")

    @functools.partial(
        pl.kernel,
        out_type=jax.ShapeDtypeStruct((NF, BATCH, EMB_DIM), jnp.float32),
        mesh=mesh,
        scratch_types=[
            pltpu.VMEM((UPW * CHUNK,), jnp.int32),
            pltpu.VMEM((CHUNK, EMB_DIM), jnp.float32),
            pltpu.VMEM((CHUNK, EMB_DIM), jnp.float32),
            pltpu.SemaphoreType.DMA,
            pltpu.SemaphoreType.DMA,
            pltpu.SemaphoreType.DMA,
            pltpu.SemaphoreType.DMA,
        ],
        compiler_params=pltpu.CompilerParams(use_tc_tiling_on_sc=False),
    )
    def k(idx_hbm, table_hbm, out_hbm, idx_v, rows0, rows1, g0, g1, o0, o1):
        wid = lax.axis_index("s") * NC + lax.axis_index("c")
        u0 = wid * UPW
        pltpu.sync_copy(idx_hbm.at[pl.ds(u0 * CHUNK, UPW * CHUNK)], idx_v)

        bufs = [(rows0, g0, o0), (rows1, g1, o1)]

        def gather(j):
            rows, gsem, _ = bufs[j % 2]
            return pltpu.async_copy(
                table_hbm.at[idx_v.at[pl.ds(j * CHUNK, CHUNK)]], rows, gsem
            )

        def writeback(j):
            rows, _, osem = bufs[j % 2]
            u = u0 + j
            f = u // NBLK
            b0 = (u % NBLK) * CHUNK
            return pltpu.async_copy(
                rows, out_hbm.at[f, pl.ds(b0, CHUNK)], osem
            )

        g_desc = {0: gather(0)}
        o_desc = {}
        for j in range(UPW):
            if j + 1 < UPW:
                if j >= 1:
                    o_desc[j - 1].wait()
                g_desc[j + 1] = gather(j + 1)
            g_desc[j].wait()
            o_desc[j] = writeback(j)
        o_desc[UPW - 2].wait()
        o_desc[UPW - 1].wait()

    return k(idx, table)


def kernel(cond_ids, emb_weight):
    idx = cond_ids.astype(jnp.int32).T.reshape(BATCH * NF)
    out = _gather_rows(idx, emb_weight)
    return (
        out.reshape(NF, BATCH // 4, 4, EMB_DIM)
        .transpose(1, 2, 0, 3)
        .reshape(BATCH, NF, EMB_DIM)
    )


# speedup vs baseline: 1.7791x; 1.0425x over previous
import functools

import jax
import jax.numpy as jnp
from jax import lax
from jax.experimental import pallas as pl
from jax.experimental.pallas import tpu as pltpu
from jax.experimental.pallas import tpu_sc as plsc

EMB_DIM = 32
BATCH = 16384
NF = 26

NC = 2
NS = 16
NW = NC * NS

CHUNK = 1024
NBLK = BATCH // CHUNK
N_UNITS = NF * NBLK
UPW = N_UNITS // NW


@jax.jit
def _gather_rows(idx, table):
    mesh = plsc.VectorSubcoreMesh(core_axis_name="c", subcore_axis_name="s")

    @functools.partial(
        pl.kernel,
        out_type=jax.ShapeDtypeStruct((NF, BATCH, EMB_DIM), jnp.float32),
        mesh=mesh,
        scratch_types=[
            pltpu.VMEM((UPW * CHUNK,), jnp.int32),
            pltpu.VMEM((CHUNK, EMB_DIM), jnp.float32),
            pltpu.VMEM((CHUNK, EMB_DIM), jnp.float32),
            pltpu.SemaphoreType.DMA,
            pltpu.SemaphoreType.DMA,
            pltpu.SemaphoreType.DMA,
            pltpu.SemaphoreType.DMA,
        ],
        compiler_params=pltpu.CompilerParams(use_tc_tiling_on_sc=False),
    )
    def k(idx_hbm, table_hbm, out_hbm, idx_v, rows0, rows1, g0, g1, o0, o1):
        wid = lax.axis_index("s") * NC + lax.axis_index("c")
        u0 = wid * UPW
        pltpu.sync_copy(idx_hbm.at[pl.ds(u0 * CHUNK, UPW * CHUNK)], idx_v)

        bufs = [(rows0, g0, o0), (rows1, g1, o1)]

        def gather(j):
            rows, gsem, _ = bufs[j % 2]
            return pltpu.async_copy(
                table_hbm.at[idx_v.at[pl.ds(j * CHUNK, CHUNK)]], rows, gsem
            )

        def writeback(j):
            rows, _, osem = bufs[j % 2]
            u = u0 + j
            f = u // NBLK
            b0 = (u % NBLK) * CHUNK
            return pltpu.async_copy(
                rows, out_hbm.at[f, pl.ds(b0, CHUNK)], osem
            )

        g_desc = {0: gather(0)}
        o_desc = {}
        for j in range(UPW):
            if j + 1 < UPW:
                if j >= 1:
                    o_desc[j - 1].wait()
                g_desc[j + 1] = gather(j + 1)
            g_desc[j].wait()
            o_desc[j] = writeback(j)
        o_desc[UPW - 2].wait()
        o_desc[UPW - 1].wait()

    return k(idx, table)


def kernel(cond_ids, emb_weight):
    idx = cond_ids.astype(jnp.int32).T.reshape(BATCH * NF) * 4
    table_p = jnp.pad(emb_weight, ((0, 0), (0, 96))).reshape(4000000, EMB_DIM)
    out = _gather_rows(idx, table_p)
    return out.transpose(1, 0, 2)
